# baseline (speedup 1.0000x reference)
import jax
import jax.numpy as jnp
from jax import lax
from jax.experimental import pallas as pl
from jax.experimental.pallas import tpu as pltpu
from jax.experimental.pallas import tpu_sc as plsc

N_PTS = 2097152
GRID = 256
WORDS = GRID * GRID * GRID // 4

NC = 2
NS = 16
NW = NC * NS

PTS_PER_W = N_PTS // NW
C = 4096
CHUNKS = PTS_PER_W // C
ROWS = C // 128


def _body(code_hbm, table_hbm, out_hbm,
          code_v, widx_v, words_v, out_v,
          csem0, csem1, gsem0, gsem1, osem0, osem1):
    wid = lax.axis_index("s") * NC + lax.axis_index("c")
    base = wid * PTS_PER_W
    lanes = lax.iota(jnp.int32, 16)
    csem = (csem0, csem1)
    gsem = (gsem0, gsem1)
    osem = (osem0, osem1)

    def code_dma(g, b):
        return pltpu.async_copy(code_hbm.at[pl.ds(base + g * C, C)],
                                code_v.at[pl.ds(b * C, C)], csem[b])

    def pass1(b):
        off = b * C

        def step(i, _):
            rows = off + i * 16 + lanes
            code = plsc.load_gather(code_v, [rows])
            plsc.store_scatter(widx_v, [rows], code & 0x7FFFFF)
            return 0

        lax.fori_loop(0, C // 16, step, 0)

    def fire_gathers(b):
        return [
            pltpu.async_copy(
                table_hbm.at[widx_v.at[pl.ds(b * C + j * 128, 128)]],
                words_v.at[pl.ds(b * C + j * 128, 128)], gsem[b])
            for j in range(ROWS)
        ]

    def pass2(b):
        off = b * C

        def step(i, _):
            rows = off + i * 16 + lanes
            wvec = plsc.load_gather(words_v, [rows])
            code = plsc.load_gather(code_v, [rows])
            sh = ((code >> 24) & 3) << 3
            val = (wvec >> sh) & 1 & (code >> 28)
            plsc.store_scatter(out_v, [rows], val)
            return 0

        lax.fori_loop(0, C // 16, step, 0)

    def out_dma(g, b):
        return pltpu.async_copy(out_v.at[pl.ds(b * C, C)],
                                out_hbm.at[pl.ds(base + g * C, C)], osem[b])

    cd = [None, None]
    gd = [None, None]
    od = [None, None]

    cd[0] = code_dma(0, 0)
    cd[0].wait()
    pass1(0)
    gd[0] = fire_gathers(0)
    cd[1] = code_dma(1, 1)

    for g in range(1, CHUNKS):
        b = g & 1
        pb = b ^ 1
        cd[b].wait()
        pass1(b)
        gd[b] = fire_gathers(b)
        for d in gd[pb]:
            d.wait()
        if od[pb] is not None:
            od[pb].wait()
        pass2(pb)
        od[pb] = out_dma(g - 1, pb)
        if g + 1 < CHUNKS:
            cd[pb] = code_dma(g + 1, pb)

    b = (CHUNKS - 1) & 1
    for d in gd[b]:
        d.wait()
    if od[b] is not None:
        od[b].wait()
    pass2(b)
    od[b] = out_dma(CHUNKS - 1, b)
    od[0].wait()
    od[1].wait()


@jax.jit
def kernel(xyz, mask, xyz2ijk_scale, xyz2ijk_shift):
    v = xyz * xyz2ijk_scale + xyz2ijk_shift
    ijk = jnp.round(v).astype(jnp.int32)
    valid = jnp.all((ijk >= 0) & (ijk < GRID), axis=-1).astype(jnp.int32)
    ic = jnp.clip(ijk, 0, GRID - 1)
    i_, j_, k_ = ic[:, 0], ic[:, 1], ic[:, 2]
    r = (i_ << 7) | ((j_ >> 2) << 1) | (k_ >> 7)
    w = (r << 7) | (k_ & 127)
    code = w | ((j_ & 3) << 24) | (valid << 28)

    m2 = mask.reshape(GRID * GRID, GRID)
    p = [m2[b::4, :].astype(jnp.uint32) for b in range(4)]
    word = p[0] | (p[1] << 8) | (p[2] << 16) | (p[3] << 24)
    table = lax.bitcast_convert_type(word, jnp.int32).reshape(WORDS)

    mesh = plsc.VectorSubcoreMesh(
        core_axis_name="c", subcore_axis_name="s",
        num_cores=NC, num_subcores=NS)
    out = pl.kernel(
        _body,
        out_type=jax.ShapeDtypeStruct((N_PTS,), jnp.int32),
        mesh=mesh,
        compiler_params=pltpu.CompilerParams(needs_layout_passes=False),
        scratch_types=[
            pltpu.VMEM((2 * C,), jnp.int32),
            pltpu.VMEM((2 * C,), jnp.int32),
            pltpu.VMEM((2 * C,), jnp.int32),
            pltpu.VMEM((2 * C,), jnp.int32),
            pltpu.SemaphoreType.DMA,
            pltpu.SemaphoreType.DMA,
            pltpu.SemaphoreType.DMA,
            pltpu.SemaphoreType.DMA,
            pltpu.SemaphoreType.DMA,
            pltpu.SemaphoreType.DMA,
        ],
    )(code, table)
    return out != 0


# speedup vs baseline: 2.0465x; 2.0465x over previous
import jax
import jax.numpy as jnp
from jax import lax
from jax.experimental import pallas as pl
from jax.experimental.pallas import tpu as pltpu
from jax.experimental.pallas import tpu_sc as plsc

N_PTS = 2097152
GRID = 256
WORDS = GRID * GRID * GRID // 4

NC = 2
NS = 16
NW = NC * NS

PTS_PER_W = N_PTS // NW
C = 4096
CHUNKS = PTS_PER_W // C
ROWS = C // 128


def _body(code_hbm, table_hbm, out_hbm,
          code_v, widx_v, words_v, out_v,
          csem0, csem1, gsem0, gsem1, osem0, osem1):
    wid = lax.axis_index("s") * NC + lax.axis_index("c")
    base = wid * PTS_PER_W
    lanes = lax.iota(jnp.int32, 16)
    csem = (csem0, csem1)
    gsem = (gsem0, gsem1)
    osem = (osem0, osem1)

    def code_dma(g, b):
        return pltpu.async_copy(code_hbm.at[pl.ds(base + g * C, C)],
                                code_v.at[pl.ds(b * C, C)], csem[b])

    def pass1(b):
        off = b * C

        def step(i, _):
            rows = off + i * 16 + lanes
            code = plsc.load_gather(code_v, [rows])
            plsc.store_scatter(widx_v, [rows], code & 0x7FFFFF)
            return 0

        lax.fori_loop(0, C // 16, step, 0)

    def fire_gathers(b):
        return [
            pltpu.async_copy(
                table_hbm.at[widx_v.at[pl.ds(b * C + j * 128, 128)]],
                words_v.at[pl.ds(b * C + j * 128, 128)], gsem[b])
            for j in range(ROWS)
        ]

    def pass2(b):
        off = b * C

        def step(i, _):
            rows = off + i * 16 + lanes
            wvec = plsc.load_gather(words_v, [rows])
            code = plsc.load_gather(code_v, [rows])
            sh = ((code >> 24) & 3) << 3
            val = (wvec >> sh) & 1 & (code >> 28)
            plsc.store_scatter(out_v, [rows], val)
            return 0

        lax.fori_loop(0, C // 16, step, 0)

    def out_dma(g, b):
        return pltpu.async_copy(out_v.at[pl.ds(b * C, C)],
                                out_hbm.at[pl.ds(base + g * C, C)], osem[b])

    cd = [None, None]
    gd = [None, None]
    od = [None, None]

    cd[0] = code_dma(0, 0)
    cd[0].wait()
    pass1(0)
    gd[0] = fire_gathers(0)
    cd[1] = code_dma(1, 1)

    for g in range(1, CHUNKS):
        b = g & 1
        pb = b ^ 1
        cd[b].wait()
        pass1(b)
        gd[b] = fire_gathers(b)
        for d in gd[pb]:
            d.wait()
        if od[pb] is not None:
            od[pb].wait()
        pass2(pb)
        od[pb] = out_dma(g - 1, pb)
        if g + 1 < CHUNKS:
            cd[pb] = code_dma(g + 1, pb)

    b = (CHUNKS - 1) & 1
    for d in gd[b]:
        d.wait()
    if od[b] is not None:
        od[b].wait()
    pass2(b)
    od[b] = out_dma(CHUNKS - 1, b)
    od[0].wait()
    od[1].wait()


@jax.jit
def kernel(xyz, mask, xyz2ijk_scale, xyz2ijk_shift):
    v = xyz * xyz2ijk_scale + xyz2ijk_shift
    ijk = jnp.round(v).astype(jnp.int32)
    valid = jnp.all((ijk >= 0) & (ijk < GRID), axis=-1).astype(jnp.int32)
    ic = jnp.clip(ijk, 0, GRID - 1)
    i_, j_, k_ = ic[:, 0], ic[:, 1], ic[:, 2]
    r = (i_ << 7) | ((k_ >> 7) << 6) | (j_ >> 2)
    w = (r << 7) | (k_ & 127)
    code = w | ((j_ & 3) << 24) | (valid << 28)

    q = mask.reshape(GRID, 64, 4, 2, 128)
    p = [q[:, :, b, :, :].astype(jnp.uint32) for b in range(4)]
    word = p[0] | (p[1] << 8) | (p[2] << 16) | (p[3] << 24)
    table = lax.bitcast_convert_type(
        jnp.transpose(word, (0, 2, 1, 3)), jnp.int32).reshape(WORDS)

    mesh = plsc.VectorSubcoreMesh(
        core_axis_name="c", subcore_axis_name="s",
        num_cores=NC, num_subcores=NS)
    out = pl.kernel(
        _body,
        out_type=jax.ShapeDtypeStruct((N_PTS,), jnp.int32),
        mesh=mesh,
        compiler_params=pltpu.CompilerParams(needs_layout_passes=False),
        scratch_types=[
            pltpu.VMEM((2 * C,), jnp.int32),
            pltpu.VMEM((2 * C,), jnp.int32),
            pltpu.VMEM((2 * C,), jnp.int32),
            pltpu.VMEM((2 * C,), jnp.int32),
            pltpu.SemaphoreType.DMA,
            pltpu.SemaphoreType.DMA,
            pltpu.SemaphoreType.DMA,
            pltpu.SemaphoreType.DMA,
            pltpu.SemaphoreType.DMA,
            pltpu.SemaphoreType.DMA,
        ],
    )(code, table)
    return out != 0


# speedup vs baseline: 2.4315x; 1.1881x over previous
import jax
import jax.numpy as jnp
from jax import lax
from jax.experimental import pallas as pl
from jax.experimental.pallas import tpu as pltpu
from jax.experimental.pallas import tpu_sc as plsc

N_PTS = 2097152
GRID = 256
WORDS = GRID * GRID * GRID // 4

NC = 2
NS = 16
NW = NC * NS

PTS_PER_W = N_PTS // NW
C = 4096
CHUNKS = PTS_PER_W // C
ROWS = C // 128


def _body(code_hbm, table_hbm, out_hbm,
          code_v, widx_v, words_v, out_v,
          csem0, csem1, gsem0, gsem1, osem0, osem1):
    wid = lax.axis_index("s") * NC + lax.axis_index("c")
    base = wid * PTS_PER_W
    lanes = lax.iota(jnp.int32, 16)
    csem = (csem0, csem1)
    gsem = (gsem0, gsem1)
    osem = (osem0, osem1)

    def code_dma(g, b):
        return pltpu.async_copy(code_hbm.at[pl.ds(base + g * C, C)],
                                code_v.at[pl.ds(b * C, C)], csem[b])

    def pass1(b):
        off = b * C

        def step(i, _):
            rows = off + i * 16 + lanes
            code = plsc.load_gather(code_v, [rows])
            plsc.store_scatter(widx_v, [rows], code & 0x3FFFFF)
            return 0

        lax.fori_loop(0, C // 16, step, 0)

    def fire_gathers(b):
        return [
            pltpu.async_copy(
                table_hbm.at[widx_v.at[pl.ds(b * C + j * 128, 128)]],
                words_v.at[pl.ds(b * C + j * 128, 128)], gsem[b])
            for j in range(ROWS)
        ]

    def pass2(b):
        off = b * C

        def step(i, _):
            rows = off + i * 16 + lanes
            wvec = plsc.load_gather(words_v, [rows])
            code = plsc.load_gather(code_v, [rows])
            sh = ((code >> 24) & 3) << 3
            ok = ((code >> 29) == 0).astype(jnp.int32)
            val = (wvec >> sh) & 1 & ok
            plsc.store_scatter(out_v, [rows], val)
            return 0

        lax.fori_loop(0, C // 16, step, 0)

    def out_dma(g, b):
        return pltpu.async_copy(out_v.at[pl.ds(b * C, C)],
                                out_hbm.at[pl.ds(base + g * C, C)], osem[b])

    cd = [None, None]
    gd = [None, None]
    od = [None, None]

    cd[0] = code_dma(0, 0)
    cd[0].wait()
    pass1(0)
    gd[0] = fire_gathers(0)
    cd[1] = code_dma(1, 1)

    for g in range(1, CHUNKS):
        b = g & 1
        pb = b ^ 1
        cd[b].wait()
        pass1(b)
        gd[b] = fire_gathers(b)
        for d in gd[pb]:
            d.wait()
        if od[pb] is not None:
            od[pb].wait()
        pass2(pb)
        od[pb] = out_dma(g - 1, pb)
        if g + 1 < CHUNKS:
            cd[pb] = code_dma(g + 1, pb)

    b = (CHUNKS - 1) & 1
    for d in gd[b]:
        d.wait()
    if od[b] is not None:
        od[b].wait()
    pass2(b)
    od[b] = out_dma(CHUNKS - 1, b)
    od[0].wait()
    od[1].wait()


@jax.jit
def kernel(xyz, mask, xyz2ijk_scale, xyz2ijk_shift):
    v = xyz * xyz2ijk_scale + xyz2ijk_shift
    r = jnp.round(v)
    oob = ((r < 0) | (r > GRID - 1)).astype(jnp.int32) << 29
    rc = jnp.clip(r, 0, GRID - 1).astype(jnp.int32)
    col = lax.broadcasted_iota(jnp.int32, (1, 3), 1)
    f0 = rc << 14
    f1 = ((rc >> 2) << 7) | ((rc & 3) << 24)
    f2 = ((rc >> 7) << 13) | (rc & 127)
    t = jnp.where(col == 0, f0, jnp.where(col == 1, f1, f2)) + oob
    code = jnp.sum(t, axis=-1)

    q = mask.reshape(GRID, 64, 4, 2, 128)
    p = [q[:, :, b, :, :].astype(jnp.uint32) for b in range(4)]
    word = p[0] | (p[1] << 8) | (p[2] << 16) | (p[3] << 24)
    table = lax.bitcast_convert_type(
        jnp.transpose(word, (0, 2, 1, 3)), jnp.int32).reshape(WORDS)

    mesh = plsc.VectorSubcoreMesh(
        core_axis_name="c", subcore_axis_name="s",
        num_cores=NC, num_subcores=NS)
    out = pl.kernel(
        _body,
        out_type=jax.ShapeDtypeStruct((N_PTS,), jnp.int32),
        mesh=mesh,
        compiler_params=pltpu.CompilerParams(needs_layout_passes=False),
        scratch_types=[
            pltpu.VMEM((2 * C,), jnp.int32),
            pltpu.VMEM((2 * C,), jnp.int32),
            pltpu.VMEM((2 * C,), jnp.int32),
            pltpu.VMEM((2 * C,), jnp.int32),
            pltpu.SemaphoreType.DMA,
            pltpu.SemaphoreType.DMA,
            pltpu.SemaphoreType.DMA,
            pltpu.SemaphoreType.DMA,
            pltpu.SemaphoreType.DMA,
            pltpu.SemaphoreType.DMA,
        ],
    )(code, table)
    return out != 0
